# baseline (speedup 1.0000x reference)
import jax
import jax.numpy as jnp
from jax import lax
from jax.experimental import pallas as pl
from jax.experimental.pallas import tpu as pltpu
from jax.experimental.pallas import tpu_sc as plsc

B, S, D = 4, 8192, 768
POOL, LEN, TOPK = 1024, 8, 8

_RB = 64
_OB = 3 * _RB
_NSTEP = (TOPK * LEN + S) // _OB
_L = 16


def _copy_score_body(xa, xb, xc, pk, big, sim_out, acc):
    t = pl.program_id(0)
    big[:, 0:_RB, :] = xa[...]
    big[:, _RB:2 * _RB, :] = xb[...]
    big[:, 2 * _RB:3 * _RB, :] = xc[...]

    s = jnp.sum(xb[...], axis=1) + jnp.sum(xc[...], axis=1)

    @pl.when(t == 0)
    def _():
        acc[...] = s

    @pl.when(t > 0)
    def _():
        acc[...] = acc[...] + s + jnp.sum(xa[...], axis=1)

    @pl.when(t == _NSTEP - 1)
    def _():
        xm = acc[...] * (1.0 / S)
        xn = xm / jnp.maximum(
            jnp.sqrt(jnp.sum(xm * xm, axis=1, keepdims=True)), 1e-12)
        pkv = pk[...]
        pk_inv = 1.0 / jnp.maximum(
            jnp.sqrt(jnp.sum(pkv * pkv, axis=1)), 1e-12)
        g = lax.dot_general(xn, pkv, (((1,), (1,)), ((), ())),
                            preferred_element_type=jnp.float32)
        sim_out[...] = g * pk_inv[None, :]


def _sc_topk_gather_body(sim_hbm, prompt_hbm, head_hbm, vals_hbm,
                         sim_v, idx_v, vals_v, rows_v, sem):
    c = lax.axis_index("c")
    s = lax.axis_index("s")
    wid = s * 2 + c

    @pl.when(wid < B)
    def _():
        b = wid
        pltpu.sync_copy(sim_hbm.at[pl.ds(b * POOL, POOL)], sim_v)
        lanes = lax.iota(jnp.int32, _L)
        neg = jnp.float32(-jnp.inf)
        idxcol = jnp.zeros((_L,), jnp.int32)
        valcol = jnp.zeros((_L,), jnp.float32)
        for k in range(TOPK):
            def chunk_body(i, carry):
                vacc, iacc = carry
                cvec = sim_v[pl.ds(i * _L, _L)]
                ci = lanes + i * _L
                gt = cvec > vacc
                return jnp.where(gt, cvec, vacc), jnp.where(gt, ci, iacc)

            vacc, iacc = lax.fori_loop(
                0, POOL // _L, chunk_body,
                (jnp.full((_L,), neg), jnp.zeros((_L,), jnp.int32)))
            m = jnp.max(vacc)
            wi = jnp.min(jnp.where(vacc == m, iacc, 2 * POOL))
            idxcol = jnp.where(lanes == k, wi, idxcol)
            valcol = jnp.where(lanes == k, m, valcol)
            plsc.store_scatter(sim_v, [jnp.full((_L,), wi, jnp.int32)],
                               jnp.full((_L,), neg), mask=lanes == 0)
        idx_v[...] = idxcol
        vals_v[...] = valcol
        pltpu.sync_copy(vals_v, vals_hbm.at[pl.ds(_L * b, _L)])
        pltpu.async_copy(prompt_hbm.at[idx_v.at[pl.ds(0, TOPK)]],
                         rows_v, sem).wait()
        pltpu.sync_copy(rows_v, head_hbm.at[pl.ds(TOPK * b, TOPK)])


def _paste_body(head, vals, big_any, out_blk, rsum):
    del big_any
    out_blk[...] = head[...]
    rsum[...] = jnp.reshape(jnp.sum(vals[...]) * (1.0 / B), (1, 1))


def kernel(x_embed, prompt, prompt_key):
    n_out_rows = TOPK * LEN + S

    big, sim = pl.pallas_call(
        _copy_score_body,
        grid=(_NSTEP,),
        in_specs=[
            pl.BlockSpec((B, _RB, D), lambda t: (0, jnp.maximum(3 * t - 1, 0), 0)),
            pl.BlockSpec((B, _RB, D), lambda t: (0, 3 * t, 0)),
            pl.BlockSpec((B, _RB, D), lambda t: (0, 3 * t + 1, 0)),
            pl.BlockSpec((POOL, D), lambda t: (0, 0)),
        ],
        out_specs=[
            pl.BlockSpec((B, _OB, D), lambda t: (0, t, 0)),
            pl.BlockSpec((B, POOL), lambda t: (0, 0)),
        ],
        out_shape=[
            jax.ShapeDtypeStruct((B, n_out_rows, D), jnp.float32),
            jax.ShapeDtypeStruct((B, POOL), jnp.float32),
        ],
        scratch_shapes=[pltpu.VMEM((B, D), jnp.float32)],
        compiler_params=pltpu.CompilerParams(
            dimension_semantics=("arbitrary",)),
    )(x_embed, x_embed, x_embed, prompt_key)

    sc_stage = pl.kernel(
        _sc_topk_gather_body,
        out_type=[
            jax.ShapeDtypeStruct((B * TOPK, LEN * D), jnp.float32),
            jax.ShapeDtypeStruct((B * _L,), jnp.float32),
        ],
        mesh=plsc.VectorSubcoreMesh(core_axis_name="c", subcore_axis_name="s"),
        scratch_types=[
            pltpu.VMEM((POOL,), jnp.float32),
            pltpu.VMEM((_L,), jnp.int32),
            pltpu.VMEM((_L,), jnp.float32),
            pltpu.VMEM((TOPK, LEN * D), jnp.float32),
            pltpu.SemaphoreType.DMA,
        ],
        compiler_params=pltpu.CompilerParams(needs_layout_passes=False),
    )
    head, vals = sc_stage(sim.reshape(B * POOL), prompt.reshape(POOL, LEN * D))

    out, rsum = pl.pallas_call(
        _paste_body,
        grid=(1,),
        in_specs=[
            pl.BlockSpec((B, TOPK * LEN, D), lambda t: (0, 0, 0)),
            pl.BlockSpec((1, B * _L), lambda t: (0, 0)),
            pl.BlockSpec(memory_space=pl.ANY),
        ],
        out_specs=[
            pl.BlockSpec((B, TOPK * LEN, D), lambda t: (0, 0, 0)),
            pl.BlockSpec((1, 1), lambda t: (0, 0)),
        ],
        out_shape=[
            jax.ShapeDtypeStruct((B, n_out_rows, D), jnp.float32),
            jax.ShapeDtypeStruct((1, 1), jnp.float32),
        ],
        input_output_aliases={2: 0},
    )(head.reshape(B, TOPK * LEN, D), vals.reshape(1, B * _L), big)

    return out, rsum[0, 0]


# speedup vs baseline: 1.0101x; 1.0101x over previous
import jax
import jax.numpy as jnp
from jax import lax
from jax.experimental import pallas as pl
from jax.experimental.pallas import tpu as pltpu
from jax.experimental.pallas import tpu_sc as plsc

B, S, D = 4, 8192, 768
POOL, LEN, TOPK = 1024, 8, 8

_RB = 64
_OB = 3 * _RB
_NSTEP = (TOPK * LEN + S) // _OB
_L = 16


def _copy_score_body(xa, xb, xc, pk, big, sim_out, acc):
    t = pl.program_id(0)
    big[:, 0:_RB, :] = xa[...]
    big[:, _RB:2 * _RB, :] = xb[...]
    big[:, 2 * _RB:3 * _RB, :] = xc[...]

    s = jnp.sum(xb[...], axis=1) + jnp.sum(xc[...], axis=1)

    @pl.when(t == 0)
    def _():
        acc[...] = s

    @pl.when(t > 0)
    def _():
        acc[...] = acc[...] + s + jnp.sum(xa[...], axis=1)

    @pl.when(t == _NSTEP - 1)
    def _():
        xm = acc[...] * (1.0 / S)
        xn = xm / jnp.maximum(
            jnp.sqrt(jnp.sum(xm * xm, axis=1, keepdims=True)), 1e-12)
        pkv = pk[...]
        pk_inv = 1.0 / jnp.maximum(
            jnp.sqrt(jnp.sum(pkv * pkv, axis=1)), 1e-12)
        g = lax.dot_general(xn, pkv, (((1,), (1,)), ((), ())),
                            preferred_element_type=jnp.float32)
        sim_out[...] = g * pk_inv[None, :]


_SEG = POOL // 8


def _sc_topk_gather_body(sim_hbm, prompt_hbm, head_hbm, vals_hbm,
                         seg_v, cval_v, cidx_v, stage_v, stage_i,
                         idx_v, vals_v, rows_v, sh_val, sh_idx, sem):
    c = lax.axis_index("c")
    s = lax.axis_index("s")
    row_local = s // 8
    b = c * 2 + row_local
    seg = s % 8
    lanes = lax.iota(jnp.int32, _L)
    neg = jnp.float32(-jnp.inf)

    pltpu.sync_copy(sim_hbm.at[pl.ds(b * POOL + seg * _SEG, _SEG)], seg_v)
    idxcol = jnp.zeros((_L,), jnp.int32)
    valcol = jnp.full((_L,), neg)
    for k in range(TOPK):
        vacc = jnp.full((_L,), neg)
        iacc = jnp.zeros((_L,), jnp.int32)
        for j in range(_SEG // _L):
            cvec = seg_v[pl.ds(j * _L, _L)]
            ci = lanes + j * _L
            gt = cvec > vacc
            vacc = jnp.where(gt, cvec, vacc)
            iacc = jnp.where(gt, ci, iacc)
        m = jnp.max(vacc)
        wi = jnp.min(jnp.where(vacc == m, iacc, 2 * POOL))
        idxcol = jnp.where(lanes == k, seg * _SEG + wi, idxcol)
        valcol = jnp.where(lanes == k, m, valcol)
        plsc.store_scatter(seg_v, [jnp.full((_L,), wi, jnp.int32)],
                           jnp.full((_L,), neg), mask=lanes == 0)
    stage_v[...] = valcol
    stage_i[...] = idxcol
    off = row_local * 8 * _L + seg * _L
    pltpu.sync_copy(stage_v, sh_val.at[pl.ds(off, _L)])
    pltpu.sync_copy(stage_i, sh_idx.at[pl.ds(off, _L)])
    plsc.subcore_barrier()

    @pl.when(seg == 0)
    def _():
        pltpu.sync_copy(sh_val.at[pl.ds(row_local * 8 * _L, 8 * _L)], cval_v)
        pltpu.sync_copy(sh_idx.at[pl.ds(row_local * 8 * _L, 8 * _L)], cidx_v)
        idxm = jnp.zeros((_L,), jnp.int32)
        valm = jnp.zeros((_L,), jnp.float32)
        for k in range(TOPK):
            vacc = jnp.full((_L,), neg)
            iacc = jnp.zeros((_L,), jnp.int32)
            for j in range(8 * _L // _L):
                cvec = cval_v[pl.ds(j * _L, _L)]
                give = cidx_v[pl.ds(j * _L, _L)]
                gt = cvec > vacc
                vacc = jnp.where(gt, cvec, vacc)
                iacc = jnp.where(gt, give, iacc)
            m = jnp.max(vacc)
            wi = jnp.min(jnp.where(vacc == m, iacc, 2 * POOL))
            idxm = jnp.where(lanes == k, wi, idxm)
            valm = jnp.where(lanes == k, m, valm)
            for j in range(8 * _L // _L):
                cvec = cval_v[pl.ds(j * _L, _L)]
                give = cidx_v[pl.ds(j * _L, _L)]
                cval_v[pl.ds(j * _L, _L)] = jnp.where(give == wi, neg, cvec)
        idx_v[...] = idxm
        vals_v[...] = valm
        pltpu.sync_copy(vals_v, vals_hbm.at[pl.ds(_L * b, _L)])
        pltpu.async_copy(prompt_hbm.at[idx_v.at[pl.ds(0, TOPK)]],
                         rows_v, sem).wait()
        pltpu.sync_copy(rows_v, head_hbm.at[pl.ds(TOPK * b, TOPK)])


def _paste_body(head, vals, big_any, out_blk, rsum):
    del big_any
    out_blk[...] = head[...]
    rsum[...] = jnp.reshape(jnp.sum(vals[...]) * (1.0 / B), (1, 1))


def kernel(x_embed, prompt, prompt_key):
    n_out_rows = TOPK * LEN + S

    big, sim = pl.pallas_call(
        _copy_score_body,
        grid=(_NSTEP,),
        in_specs=[
            pl.BlockSpec((B, _RB, D), lambda t: (0, jnp.maximum(3 * t - 1, 0), 0)),
            pl.BlockSpec((B, _RB, D), lambda t: (0, 3 * t, 0)),
            pl.BlockSpec((B, _RB, D), lambda t: (0, 3 * t + 1, 0)),
            pl.BlockSpec((POOL, D), lambda t: (0, 0)),
        ],
        out_specs=[
            pl.BlockSpec((B, _OB, D), lambda t: (0, t, 0)),
            pl.BlockSpec((B, POOL), lambda t: (0, 0)),
        ],
        out_shape=[
            jax.ShapeDtypeStruct((B, n_out_rows, D), jnp.float32),
            jax.ShapeDtypeStruct((B, POOL), jnp.float32),
        ],
        scratch_shapes=[pltpu.VMEM((B, D), jnp.float32)],
        compiler_params=pltpu.CompilerParams(
            dimension_semantics=("arbitrary",)),
    )(x_embed, x_embed, x_embed, prompt_key)

    sc_stage = pl.kernel(
        _sc_topk_gather_body,
        out_type=[
            jax.ShapeDtypeStruct((B * TOPK, LEN * D), jnp.float32),
            jax.ShapeDtypeStruct((B * _L,), jnp.float32),
        ],
        mesh=plsc.VectorSubcoreMesh(core_axis_name="c", subcore_axis_name="s"),
        scratch_types=[
            pltpu.VMEM((_SEG,), jnp.float32),
            pltpu.VMEM((8 * _L,), jnp.float32),
            pltpu.VMEM((8 * _L,), jnp.int32),
            pltpu.VMEM((_L,), jnp.float32),
            pltpu.VMEM((_L,), jnp.int32),
            pltpu.VMEM((_L,), jnp.int32),
            pltpu.VMEM((_L,), jnp.float32),
            pltpu.VMEM((TOPK, LEN * D), jnp.float32),
            pltpu.VMEM_SHARED((2 * 8 * _L,), jnp.float32),
            pltpu.VMEM_SHARED((2 * 8 * _L,), jnp.int32),
            pltpu.SemaphoreType.DMA,
        ],
        compiler_params=pltpu.CompilerParams(needs_layout_passes=False),
    )
    head, vals = sc_stage(sim.reshape(B * POOL), prompt.reshape(POOL, LEN * D))

    out, rsum = pl.pallas_call(
        _paste_body,
        grid=(1,),
        in_specs=[
            pl.BlockSpec((B, TOPK * LEN, D), lambda t: (0, 0, 0)),
            pl.BlockSpec((1, B * _L), lambda t: (0, 0)),
            pl.BlockSpec(memory_space=pl.ANY),
        ],
        out_specs=[
            pl.BlockSpec((B, TOPK * LEN, D), lambda t: (0, 0, 0)),
            pl.BlockSpec((1, 1), lambda t: (0, 0)),
        ],
        out_shape=[
            jax.ShapeDtypeStruct((B, n_out_rows, D), jnp.float32),
            jax.ShapeDtypeStruct((1, 1), jnp.float32),
        ],
        input_output_aliases={2: 0},
    )(head.reshape(B, TOPK * LEN, D), vals.reshape(1, B * _L), big)

    return out, rsum[0, 0]


# speedup vs baseline: 1.1957x; 1.1838x over previous
import jax
import jax.numpy as jnp
from jax import lax
from jax.experimental import pallas as pl
from jax.experimental.pallas import tpu as pltpu
from jax.experimental.pallas import tpu_sc as plsc

B, S, D = 4, 8192, 768
POOL, LEN, TOPK = 1024, 8, 8

_RB = 64
_OB = 3 * _RB
_NSTEP = (TOPK * LEN + S) // _OB
_L = 16


def _copy_score_body(xa, xb, xc, pk, big, sim_out, acc):
    t = pl.program_id(0)
    big[:, 0:_RB, :] = xa[...]
    big[:, _RB:2 * _RB, :] = xb[...]
    big[:, 2 * _RB:3 * _RB, :] = xc[...]

    s = jnp.sum(xb[...], axis=1) + jnp.sum(xc[...], axis=1)

    @pl.when(t == 0)
    def _():
        acc[...] = s

    @pl.when(t > 0)
    def _():
        acc[...] = acc[...] + s + jnp.sum(xa[...], axis=1)

    @pl.when(t == _NSTEP - 1)
    def _():
        xm = acc[...] * (1.0 / S)
        xn = xm / jnp.maximum(
            jnp.sqrt(jnp.sum(xm * xm, axis=1, keepdims=True)), 1e-12)
        pkv = pk[...]
        pk_inv = 1.0 / jnp.maximum(
            jnp.sqrt(jnp.sum(pkv * pkv, axis=1)), 1e-12)
        g = lax.dot_general(xn, pkv, (((1,), (1,)), ((), ())),
                            preferred_element_type=jnp.float32)
        sim_out[...] = g * pk_inv[None, :]


_SEG = POOL // 8


def _sc_topk_gather_body(sim_hbm, prompt_hbm, head_hbm, vals_hbm,
                         seg_v, cval_v, cidx_v, stage_v, stage_i,
                         idx_v, vals_v, rows_v, sh_val, sh_idx, sem):
    c = lax.axis_index("c")
    s = lax.axis_index("s")
    row_local = s // 8
    b = c * 2 + row_local
    seg = s % 8
    lanes = lax.iota(jnp.int32, _L)
    neg = jnp.float32(-jnp.inf)

    pltpu.sync_copy(sim_hbm.at[pl.ds(b * POOL + seg * _SEG, _SEG)], seg_v)
    idxcol = jnp.zeros((_L,), jnp.int32)
    valcol = jnp.full((_L,), neg)
    for k in range(TOPK):
        vacc = jnp.full((_L,), neg)
        iacc = jnp.zeros((_L,), jnp.int32)
        for j in range(_SEG // _L):
            cvec = seg_v[pl.ds(j * _L, _L)]
            ci = lanes + j * _L
            gt = cvec > vacc
            vacc = jnp.where(gt, cvec, vacc)
            iacc = jnp.where(gt, ci, iacc)
        m = jnp.max(vacc)
        wi = jnp.min(jnp.where(vacc == m, iacc, 2 * POOL))
        idxcol = jnp.where(lanes == k, seg * _SEG + wi, idxcol)
        valcol = jnp.where(lanes == k, m, valcol)
        plsc.store_scatter(seg_v, [jnp.full((_L,), wi, jnp.int32)],
                           jnp.full((_L,), neg), mask=lanes == 0)
    stage_v[...] = valcol
    stage_i[...] = idxcol
    off = row_local * 8 * _L + seg * _L
    pltpu.sync_copy(stage_v, sh_val.at[pl.ds(off, _L)])
    pltpu.sync_copy(stage_i, sh_idx.at[pl.ds(off, _L)])
    plsc.subcore_barrier()

    @pl.when(seg == 0)
    def _():
        pltpu.sync_copy(sh_val.at[pl.ds(row_local * 8 * _L, 8 * _L)], cval_v)
        pltpu.sync_copy(sh_idx.at[pl.ds(row_local * 8 * _L, 8 * _L)], cidx_v)
        idxm = jnp.zeros((_L,), jnp.int32)
        valm = jnp.zeros((_L,), jnp.float32)
        for k in range(TOPK):
            vacc = jnp.full((_L,), neg)
            iacc = jnp.zeros((_L,), jnp.int32)
            for j in range(8 * _L // _L):
                cvec = cval_v[pl.ds(j * _L, _L)]
                give = cidx_v[pl.ds(j * _L, _L)]
                gt = cvec > vacc
                vacc = jnp.where(gt, cvec, vacc)
                iacc = jnp.where(gt, give, iacc)
            m = jnp.max(vacc)
            wi = jnp.min(jnp.where(vacc == m, iacc, 2 * POOL))
            idxm = jnp.where(lanes == k, wi, idxm)
            valm = jnp.where(lanes == k, m, valm)
            for j in range(8 * _L // _L):
                cvec = cval_v[pl.ds(j * _L, _L)]
                give = cidx_v[pl.ds(j * _L, _L)]
                cval_v[pl.ds(j * _L, _L)] = jnp.where(give == wi, neg, cvec)
        idx_v[...] = idxm
        vals_v[...] = valm
        pltpu.sync_copy(vals_v, vals_hbm.at[pl.ds(_L * b, _L)])
        pltpu.async_copy(prompt_hbm.at[idx_v.at[pl.ds(0, TOPK)]],
                         rows_v, sem).wait()
        pltpu.sync_copy(rows_v, head_hbm.at[pl.ds(TOPK * b, TOPK)])


def _paste_body(head, vals, big_any, out_blk, rsum):
    del big_any
    out_blk[...] = head[...].reshape(B, TOPK * LEN, D)
    rsum[...] = jnp.reshape(jnp.sum(vals[...]) * (1.0 / B), (1, 1))


def kernel(x_embed, prompt, prompt_key):
    n_out_rows = TOPK * LEN + S

    big, sim = pl.pallas_call(
        _copy_score_body,
        grid=(_NSTEP,),
        in_specs=[
            pl.BlockSpec((B, _RB, D), lambda t: (0, jnp.maximum(3 * t - 1, 0), 0)),
            pl.BlockSpec((B, _RB, D), lambda t: (0, 3 * t, 0)),
            pl.BlockSpec((B, _RB, D), lambda t: (0, 3 * t + 1, 0)),
            pl.BlockSpec((POOL, D), lambda t: (0, 0)),
        ],
        out_specs=[
            pl.BlockSpec((B, _OB, D), lambda t: (0, t, 0)),
            pl.BlockSpec((B, POOL), lambda t: (0, 0)),
        ],
        out_shape=[
            jax.ShapeDtypeStruct((B, n_out_rows, D), jnp.float32),
            jax.ShapeDtypeStruct((B, POOL), jnp.float32),
        ],
        scratch_shapes=[pltpu.VMEM((B, D), jnp.float32)],
        compiler_params=pltpu.CompilerParams(
            dimension_semantics=("arbitrary",)),
    )(x_embed, x_embed, x_embed, prompt_key)

    sc_stage = pl.kernel(
        _sc_topk_gather_body,
        out_type=[
            jax.ShapeDtypeStruct((B * TOPK, LEN, D), jnp.float32),
            jax.ShapeDtypeStruct((B * _L,), jnp.float32),
        ],
        mesh=plsc.VectorSubcoreMesh(core_axis_name="c", subcore_axis_name="s"),
        scratch_types=[
            pltpu.VMEM((_SEG,), jnp.float32),
            pltpu.VMEM((8 * _L,), jnp.float32),
            pltpu.VMEM((8 * _L,), jnp.int32),
            pltpu.VMEM((_L,), jnp.float32),
            pltpu.VMEM((_L,), jnp.int32),
            pltpu.VMEM((_L,), jnp.int32),
            pltpu.VMEM((_L,), jnp.float32),
            pltpu.VMEM((TOPK, LEN, D), jnp.float32),
            pltpu.VMEM_SHARED((2 * 8 * _L,), jnp.float32),
            pltpu.VMEM_SHARED((2 * 8 * _L,), jnp.int32),
            pltpu.SemaphoreType.DMA,
        ],
        compiler_params=pltpu.CompilerParams(needs_layout_passes=False),
    )
    head, vals = sc_stage(sim.reshape(B * POOL), prompt[0])

    out, rsum = pl.pallas_call(
        _paste_body,
        grid=(1,),
        in_specs=[
            pl.BlockSpec((B * TOPK, LEN, D), lambda t: (0, 0, 0)),
            pl.BlockSpec((1, B * _L), lambda t: (0, 0)),
            pl.BlockSpec(memory_space=pl.ANY),
        ],
        out_specs=[
            pl.BlockSpec((B, TOPK * LEN, D), lambda t: (0, 0, 0)),
            pl.BlockSpec((1, 1), lambda t: (0, 0)),
        ],
        out_shape=[
            jax.ShapeDtypeStruct((B, n_out_rows, D), jnp.float32),
            jax.ShapeDtypeStruct((1, 1), jnp.float32),
        ],
        input_output_aliases={2: 0},
    )(head, vals.reshape(1, B * _L), big)

    return out, rsum[0, 0]


# speedup vs baseline: 1.2361x; 1.0338x over previous
import jax
import jax.numpy as jnp
from jax import lax
from jax.experimental import pallas as pl
from jax.experimental.pallas import tpu as pltpu
from jax.experimental.pallas import tpu_sc as plsc

B, S, D = 4, 8192, 768
POOL, LEN, TOPK = 1024, 8, 8

_RB = 64
_NSUB = 8
_OB = _NSUB * _RB
_NSTEP = -(-(TOPK * LEN + S) // _OB)
_NXB = S // _RB
_L = 16


def _copy_score_body(*refs):
    xs = refs[:_NSUB]
    pk, big, sim_out, acc = refs[_NSUB:]
    t = pl.program_id(0)
    for j in range(_NSUB):
        big[:, j * _RB:(j + 1) * _RB, :] = xs[j][...]

    s = jnp.sum(xs[1][...], axis=1)
    for j in range(2, _NSUB):
        s = s + jnp.sum(xs[j][...], axis=1)

    @pl.when(t == 0)
    def _():
        acc[...] = s

    @pl.when((t > 0) & (t < _NSTEP - 1))
    def _():
        acc[...] = acc[...] + s + jnp.sum(xs[0][...], axis=1)

    @pl.when(t == _NSTEP - 1)
    def _():
        acc[...] = acc[...] + jnp.sum(xs[0][...], axis=1)
        xm = acc[...] * (1.0 / S)
        xn = xm / jnp.maximum(
            jnp.sqrt(jnp.sum(xm * xm, axis=1, keepdims=True)), 1e-12)
        pkv = pk[...]
        pk_inv = 1.0 / jnp.maximum(
            jnp.sqrt(jnp.sum(pkv * pkv, axis=1)), 1e-12)
        g = lax.dot_general(xn, pkv, (((1,), (1,)), ((), ())),
                            preferred_element_type=jnp.float32)
        sim_out[...] = g * pk_inv[None, :]


_SEG = POOL // 8


def _sc_topk_gather_body(sim_hbm, prompt_hbm, head_hbm, vals_hbm,
                         seg_v, cval_v, cidx_v, stage_v, stage_i,
                         idx_v, vals_v, rows_v, sh_val, sh_idx, sem):
    c = lax.axis_index("c")
    s = lax.axis_index("s")
    row_local = s // 8
    b = c * 2 + row_local
    seg = s % 8
    lanes = lax.iota(jnp.int32, _L)
    neg = jnp.float32(-jnp.inf)

    pltpu.sync_copy(sim_hbm.at[pl.ds(b * POOL + seg * _SEG, _SEG)], seg_v)
    idxcol = jnp.zeros((_L,), jnp.int32)
    valcol = jnp.full((_L,), neg)
    for k in range(TOPK):
        vacc = jnp.full((_L,), neg)
        iacc = jnp.zeros((_L,), jnp.int32)
        for j in range(_SEG // _L):
            cvec = seg_v[pl.ds(j * _L, _L)]
            ci = lanes + j * _L
            gt = cvec > vacc
            vacc = jnp.where(gt, cvec, vacc)
            iacc = jnp.where(gt, ci, iacc)
        m = jnp.max(vacc)
        wi = jnp.min(jnp.where(vacc == m, iacc, 2 * POOL))
        idxcol = jnp.where(lanes == k, seg * _SEG + wi, idxcol)
        valcol = jnp.where(lanes == k, m, valcol)
        plsc.store_scatter(seg_v, [jnp.full((_L,), wi, jnp.int32)],
                           jnp.full((_L,), neg), mask=lanes == 0)
    stage_v[...] = valcol
    stage_i[...] = idxcol
    off = row_local * 8 * _L + seg * _L
    pltpu.sync_copy(stage_v, sh_val.at[pl.ds(off, _L)])
    pltpu.sync_copy(stage_i, sh_idx.at[pl.ds(off, _L)])
    plsc.subcore_barrier()

    @pl.when(seg == 0)
    def _():
        pltpu.sync_copy(sh_val.at[pl.ds(row_local * 8 * _L, 8 * _L)], cval_v)
        pltpu.sync_copy(sh_idx.at[pl.ds(row_local * 8 * _L, 8 * _L)], cidx_v)
        idxm = jnp.zeros((_L,), jnp.int32)
        valm = jnp.zeros((_L,), jnp.float32)
        for k in range(TOPK):
            vacc = jnp.full((_L,), neg)
            iacc = jnp.zeros((_L,), jnp.int32)
            for j in range(8 * _L // _L):
                cvec = cval_v[pl.ds(j * _L, _L)]
                give = cidx_v[pl.ds(j * _L, _L)]
                gt = cvec > vacc
                vacc = jnp.where(gt, cvec, vacc)
                iacc = jnp.where(gt, give, iacc)
            m = jnp.max(vacc)
            wi = jnp.min(jnp.where(vacc == m, iacc, 2 * POOL))
            idxm = jnp.where(lanes == k, wi, idxm)
            valm = jnp.where(lanes == k, m, valm)
            for j in range(8 * _L // _L):
                cvec = cval_v[pl.ds(j * _L, _L)]
                give = cidx_v[pl.ds(j * _L, _L)]
                cval_v[pl.ds(j * _L, _L)] = jnp.where(give == wi, neg, cvec)
        idx_v[...] = idxm
        vals_v[...] = valm
        pltpu.sync_copy(vals_v, vals_hbm.at[pl.ds(_L * b, _L)])
        pltpu.async_copy(prompt_hbm.at[idx_v.at[pl.ds(0, TOPK)]],
                         rows_v, sem).wait()
        pltpu.sync_copy(rows_v, head_hbm.at[pl.ds(TOPK * b, TOPK)])


def _paste_body(head, vals, big_any, out_blk, rsum):
    del big_any
    out_blk[...] = head[...].reshape(B, TOPK * LEN, D)
    rsum[...] = jnp.reshape(jnp.sum(vals[...]) * (1.0 / B), (1, 1))


def kernel(x_embed, prompt, prompt_key):
    n_out_rows = TOPK * LEN + S

    big, sim = pl.pallas_call(
        _copy_score_body,
        grid=(_NSTEP,),
        in_specs=[
            pl.BlockSpec(
                (B, _RB, D),
                lambda t, j=j: (
                    0, jnp.clip(_NSUB * t + j - 1, 0, _NXB - 1), 0))
            for j in range(_NSUB)
        ] + [
            pl.BlockSpec((POOL, D), lambda t: (0, 0)),
        ],
        out_specs=[
            pl.BlockSpec((B, _OB, D), lambda t: (0, t, 0)),
            pl.BlockSpec((B, POOL), lambda t: (0, 0)),
        ],
        out_shape=[
            jax.ShapeDtypeStruct((B, n_out_rows, D), jnp.float32),
            jax.ShapeDtypeStruct((B, POOL), jnp.float32),
        ],
        scratch_shapes=[pltpu.VMEM((B, D), jnp.float32)],
        compiler_params=pltpu.CompilerParams(
            dimension_semantics=("arbitrary",)),
    )(*([x_embed] * _NSUB), prompt_key)

    sc_stage = pl.kernel(
        _sc_topk_gather_body,
        out_type=[
            jax.ShapeDtypeStruct((B * TOPK, LEN, D), jnp.float32),
            jax.ShapeDtypeStruct((B * _L,), jnp.float32),
        ],
        mesh=plsc.VectorSubcoreMesh(core_axis_name="c", subcore_axis_name="s"),
        scratch_types=[
            pltpu.VMEM((_SEG,), jnp.float32),
            pltpu.VMEM((8 * _L,), jnp.float32),
            pltpu.VMEM((8 * _L,), jnp.int32),
            pltpu.VMEM((_L,), jnp.float32),
            pltpu.VMEM((_L,), jnp.int32),
            pltpu.VMEM((_L,), jnp.int32),
            pltpu.VMEM((_L,), jnp.float32),
            pltpu.VMEM((TOPK, LEN, D), jnp.float32),
            pltpu.VMEM_SHARED((2 * 8 * _L,), jnp.float32),
            pltpu.VMEM_SHARED((2 * 8 * _L,), jnp.int32),
            pltpu.SemaphoreType.DMA,
        ],
        compiler_params=pltpu.CompilerParams(needs_layout_passes=False),
    )
    head, vals = sc_stage(sim.reshape(B * POOL), prompt[0])

    out, rsum = pl.pallas_call(
        _paste_body,
        grid=(1,),
        in_specs=[
            pl.BlockSpec((B * TOPK, LEN, D), lambda t: (0, 0, 0)),
            pl.BlockSpec((1, B * _L), lambda t: (0, 0)),
            pl.BlockSpec(memory_space=pl.ANY),
        ],
        out_specs=[
            pl.BlockSpec((B, TOPK * LEN, D), lambda t: (0, 0, 0)),
            pl.BlockSpec((1, 1), lambda t: (0, 0)),
        ],
        out_shape=[
            jax.ShapeDtypeStruct((B, n_out_rows, D), jnp.float32),
            jax.ShapeDtypeStruct((1, 1), jnp.float32),
        ],
        input_output_aliases={2: 0},
    )(head, vals.reshape(1, B * _L), big)

    return out, rsum[0, 0]
